# Initial kernel scaffold; baseline (speedup 1.0000x reference)
import functools

import jax
import jax.numpy as jnp
from jax import lax
from jax.experimental import pallas as pl
from jax.experimental.pallas import tpu as pltpu
from jax.experimental.pallas import tpu_sc as plsc

_NC, _NS, _L = 2, 16, 16
_NW = _NC * _NS


def _make_hist(n_idx, n_bins, interpret=False):
    chunk = n_idx // _NW
    assert chunk * _NW == n_idx and chunk % _L == 0 and n_bins % _L == 0
    mesh = plsc.VectorSubcoreMesh(core_axis_name="c", subcore_axis_name="s")

    @functools.partial(
        pl.kernel,
        out_type=jax.ShapeDtypeStruct((_NW, n_bins), jnp.float32),
        mesh=mesh,
        scratch_types=[
            pltpu.VMEM((chunk,), jnp.int32),
            pltpu.VMEM((n_bins,), jnp.float32),
        ],
        interpret=interpret,
    )
    def hist_kernel(idx_hbm, out_hbm, idx_v, hist_v):
        wid = lax.axis_index("s") * _NC + lax.axis_index("c")
        pltpu.sync_copy(idx_hbm.at[pl.ds(wid * chunk, chunk)], idx_v)

        zeros = jnp.zeros((_L,), jnp.float32)

        def zero_body(i, _):
            hist_v[pl.ds(i * _L, _L)] = zeros
            return 0

        lax.fori_loop(0, n_bins // _L, zero_body, 0, unroll=8)

        ones = jnp.ones((_L,), jnp.float32)

        def add_body(i, _):
            idx = idx_v[pl.ds(i * _L, _L)]
            plsc.addupdate_scatter(hist_v, [idx], ones)
            return 0

        lax.fori_loop(0, chunk // _L, add_body, 0, unroll=8)

        pltpu.sync_copy(hist_v, out_hbm.at[wid])

    return hist_kernel


_ROW_BLK = 1000


def _epilogue(partials_t, w_t, b_row, interpret=False):
    n_bins = partials_t.shape[0]
    out_f = w_t.shape[1]
    grid = n_bins // _ROW_BLK

    def body(p_ref, wt_ref, b_ref, o_ref):
        counts = jnp.sum(p_ref[...], axis=1, keepdims=True)
        v = jnp.sum(wt_ref[...], axis=0, keepdims=True) + b_ref[...]
        o_ref[...] = jnp.maximum(counts * v, 0.0)

    return pl.pallas_call(
        body,
        grid=(grid,),
        in_specs=[
            pl.BlockSpec((_ROW_BLK, _NW), lambda i: (i, 0)),
            pl.BlockSpec(w_t.shape, lambda i: (0, 0)),
            pl.BlockSpec((1, out_f), lambda i: (0, 0)),
        ],
        out_specs=pl.BlockSpec((_ROW_BLK, out_f), lambda i: (i, 0)),
        out_shape=jax.ShapeDtypeStruct((n_bins, out_f), jnp.float32),
        interpret=interpret,
    )(partials_t, w_t, b_row)


def kernel(local_entity, edge_list, W, b):
    bs, m = local_entity.shape
    n_bins = bs * m
    idx = jnp.concatenate([edge_list[0], edge_list[2]])
    partials = _make_hist(idx.shape[0], n_bins)(idx)
    out = _epilogue(partials.T, W.T, b.reshape(1, -1))
    return out.reshape(bs, m, W.shape[1])


# speedup vs baseline: 26.8454x; 26.8454x over previous
import functools

import jax
import jax.numpy as jnp
from jax import lax
from jax.experimental import pallas as pl
from jax.experimental.pallas import tpu as pltpu
from jax.experimental.pallas import tpu_sc as plsc

_NC, _NS, _L = 2, 16, 16
_NW = _NC * _NS


def _make_hist(n_idx, n_bins, interpret=False):
    chunk = n_idx // _NW
    assert chunk * _NW == n_idx and chunk % _L == 0 and n_bins % _L == 0
    mesh = plsc.VectorSubcoreMesh(
        core_axis_name="c", subcore_axis_name="s", num_cores=_NC, num_subcores=_NS
    )

    @functools.partial(
        pl.kernel,
        out_type=jax.ShapeDtypeStruct((_NW, n_bins), jnp.float32),
        mesh=mesh,
        scratch_types=[
            pltpu.VMEM((chunk,), jnp.int32),
            pltpu.VMEM((n_bins,), jnp.float32),
        ],
        compiler_params=pltpu.CompilerParams(needs_layout_passes=False),
        interpret=interpret,
    )
    def hist_kernel(idx_hbm, out_hbm, idx_v, hist_v):
        wid = lax.axis_index("s") * _NC + lax.axis_index("c")
        pltpu.sync_copy(idx_hbm.at[pl.ds(wid * chunk, chunk)], idx_v)

        zeros = jnp.zeros((_L,), jnp.float32)

        def zero_body(i, _):
            hist_v[pl.ds(i * _L, _L)] = zeros
            return 0

        lax.fori_loop(0, n_bins // _L, zero_body, 0, unroll=8)

        ones = jnp.ones((_L,), jnp.float32)

        def add_body(i, _):
            idx = idx_v[pl.ds(i * _L, _L)]
            plsc.addupdate_scatter(hist_v, [idx], ones)
            return 0

        lax.fori_loop(0, chunk // _L, add_body, 0, unroll=8)

        pltpu.sync_copy(hist_v, out_hbm.at[wid])

    return hist_kernel


_ROW_BLK = 1000


def _epilogue(partials_t, w_t, b_row, interpret=False):
    n_bins = partials_t.shape[0]
    out_f = w_t.shape[1]
    grid = n_bins // _ROW_BLK

    def body(p_ref, wt_ref, b_ref, o_ref):
        counts = jnp.sum(p_ref[...], axis=1, keepdims=True)
        v = jnp.sum(wt_ref[...], axis=0, keepdims=True) + b_ref[...]
        o_ref[...] = jnp.maximum(counts * v, 0.0)

    return pl.pallas_call(
        body,
        grid=(grid,),
        in_specs=[
            pl.BlockSpec((_ROW_BLK, _NW), lambda i: (i, 0)),
            pl.BlockSpec(w_t.shape, lambda i: (0, 0)),
            pl.BlockSpec((1, out_f), lambda i: (0, 0)),
        ],
        out_specs=pl.BlockSpec((_ROW_BLK, out_f), lambda i: (i, 0)),
        out_shape=jax.ShapeDtypeStruct((n_bins, out_f), jnp.float32),
        interpret=interpret,
    )(partials_t, w_t, b_row)


def kernel(local_entity, edge_list, W, b):
    bs, m = local_entity.shape
    n_bins = bs * m
    idx = jnp.concatenate([edge_list[0], edge_list[2]])
    partials = _make_hist(idx.shape[0], n_bins)(idx)
    out = _epilogue(partials.T, W.T, b.reshape(1, -1))
    return out.reshape(bs, m, W.shape[1])


# speedup vs baseline: 28.8919x; 1.0762x over previous
import functools

import jax
import jax.numpy as jnp
from jax import lax
from jax.experimental import pallas as pl
from jax.experimental.pallas import tpu as pltpu
from jax.experimental.pallas import tpu_sc as plsc

_NC, _NS, _L = 2, 16, 16
_NW = _NC * _NS


def _make_hist(n_edges, n_bins):
    chunk = n_edges // _NW
    per_w = 2 * chunk
    assert chunk * _NW == n_edges and chunk % 8 == 0 and per_w % _L == 0
    assert n_bins % _L == 0
    mesh = plsc.VectorSubcoreMesh(
        core_axis_name="c", subcore_axis_name="s", num_cores=_NC, num_subcores=_NS
    )

    @functools.partial(
        pl.kernel,
        out_type=jax.ShapeDtypeStruct((_NW, n_bins), jnp.float32),
        mesh=mesh,
        scratch_types=[
            pltpu.VMEM((per_w,), jnp.int32),
            pltpu.VMEM((n_bins,), jnp.float32),
            pltpu.SemaphoreType.DMA,
        ],
        compiler_params=pltpu.CompilerParams(needs_layout_passes=False),
    )
    def hist_kernel(edge_hbm, out_hbm, idx_v, hist_v, sem):
        wid = lax.axis_index("s") * _NC + lax.axis_index("c")
        base = wid * chunk
        cp_h = pltpu.make_async_copy(
            edge_hbm.at[pl.ds(base, chunk)], idx_v.at[pl.ds(0, chunk)], sem
        )
        cp_t = pltpu.make_async_copy(
            edge_hbm.at[pl.ds(2 * n_edges + base, chunk)],
            idx_v.at[pl.ds(chunk, chunk)],
            sem,
        )
        cp_h.start()
        cp_t.start()

        zeros = jnp.zeros((_L,), jnp.float32)

        def zero_body(i, _):
            hist_v[pl.ds(i * _L, _L)] = zeros
            return 0

        lax.fori_loop(0, n_bins // _L, zero_body, 0, unroll=8)

        cp_h.wait()
        cp_t.wait()

        ones = jnp.ones((_L,), jnp.float32)

        def add_body(i, _):
            idx = idx_v[pl.ds(i * _L, _L)]
            plsc.addupdate_scatter(hist_v, [idx], ones)
            return 0

        lax.fori_loop(0, per_w // _L, add_body, 0, unroll=8)

        pltpu.sync_copy(hist_v, out_hbm.at[wid])

    return hist_kernel


_ROW_BLK = 1000


def _epilogue(partials4, w_t, b_row):
    nblk = partials4.shape[1]
    blk = partials4.shape[3]
    out_f = w_t.shape[1]
    n_bins = nblk * blk

    def body(p_ref, wt_ref, b_ref, o_ref):
        v = jnp.sum(wt_ref[...], axis=0, keepdims=True) + b_ref[...]
        vb = jnp.broadcast_to(v, (_NW, out_f))
        p = p_ref[...].reshape(_NW, blk)
        o = lax.dot_general(
            p,
            vb,
            (((0,), (0,)), ((), ())),
            preferred_element_type=jnp.float32,
            precision=lax.Precision.HIGHEST,
        )
        o_ref[...] = jnp.maximum(o, 0.0)

    return pl.pallas_call(
        body,
        grid=(nblk,),
        in_specs=[
            pl.BlockSpec((_NW, 1, 1, blk), lambda i: (0, i, 0, 0)),
            pl.BlockSpec(w_t.shape, lambda i: (0, 0)),
            pl.BlockSpec((1, out_f), lambda i: (0, 0)),
        ],
        out_specs=pl.BlockSpec((_ROW_BLK, out_f), lambda i: (i, 0)),
        out_shape=jax.ShapeDtypeStruct((n_bins, out_f), jnp.float32),
    )(partials4, w_t, b_row)


def kernel(local_entity, edge_list, W, b):
    bs, m = local_entity.shape
    n_bins = bs * m
    n_edges = edge_list.shape[1]
    partials = _make_hist(n_edges, n_bins)(edge_list.reshape(-1))
    partials4 = partials.reshape(_NW, n_bins // _ROW_BLK, 1, _ROW_BLK)
    out = _epilogue(partials4, W.T, b.reshape(1, -1))
    return out.reshape(bs, m, W.shape[1])
